# baseline (speedup 1.0000x reference)
import jax
import jax.numpy as jnp
from jax.experimental import pallas as pl
from jax.experimental.pallas import tpu as pltpu

_NE = 8
_D = 1024
_H = 4096
_BH = 512
_BM = 1024


def _gelu_exact(v):
    return 0.5 * v * (1.0 + jax.lax.erf(v * 0.7071067811865476))


def _moe_kernel(x_ref, xb_ref, Wg_ref, bg_ref, W1_ref, b1_ref, W2_ref, b2_ref,
                out_ref, mask_ref):
    e = pl.program_id(1)
    h = pl.program_id(2)

    @pl.when((e == 0) & (h == 0))
    def _init():
        scores = jnp.dot(x_ref[...], Wg_ref[...],
                         preferred_element_type=jnp.float32) + bg_ref[...]
        lane = jax.lax.broadcasted_iota(jnp.int32, scores.shape, 1)
        m1 = jnp.max(scores, axis=-1, keepdims=True)
        first1 = jnp.min(jnp.where(scores == m1, lane, _NE), axis=-1,
                         keepdims=True)
        rest = jnp.where(lane == first1, -jnp.inf, scores)
        m2 = jnp.max(rest, axis=-1, keepdims=True)
        first2 = jnp.min(jnp.where(rest == m2, lane, _NE), axis=-1,
                         keepdims=True)
        mask_ref[...] = ((lane == first1) | (lane == first2)).astype(
            jnp.float32)
        out_ref[...] = jnp.zeros_like(out_ref)

    lane = jax.lax.broadcasted_iota(jnp.int32, mask_ref.shape, 1)
    mcol = jnp.sum(jnp.where(lane == e, mask_ref[...], 0.0), axis=-1,
                   keepdims=True)

    hblk = jnp.dot(xb_ref[...], W1_ref[0],
                   preferred_element_type=jnp.float32) + b1_ref[0]
    hblk = (_gelu_exact(hblk) * mcol).astype(jnp.bfloat16)
    out_ref[...] += jnp.dot(hblk, W2_ref[0],
                            preferred_element_type=jnp.float32)

    @pl.when(h == 0)
    def _bias2():
        out_ref[...] += mcol * b2_ref[0]


def kernel(x, W1, b1, W2, b2, Wg, bg):
    B, S, D = x.shape
    T = B * S
    x2 = x.reshape(T, D)
    xb = x2.astype(jnp.bfloat16)
    W1b = W1.astype(jnp.bfloat16)
    W2b = W2.astype(jnp.bfloat16)
    bg2 = bg.reshape(1, _NE)
    b1_3 = b1.reshape(_NE, 1, _H)
    b2_3 = b2.reshape(_NE, 1, D)

    nh = _H // _BH
    nm = T // _BM
    out = pl.pallas_call(
        _moe_kernel,
        grid=(nm, _NE, nh),
        in_specs=[
            pl.BlockSpec((_BM, D), lambda m, e, h: (m, 0)),
            pl.BlockSpec((_BM, D), lambda m, e, h: (m, 0)),
            pl.BlockSpec((D, _NE), lambda m, e, h: (0, 0)),
            pl.BlockSpec((1, _NE), lambda m, e, h: (0, 0)),
            pl.BlockSpec((1, D, _BH), lambda m, e, h: (e, 0, h)),
            pl.BlockSpec((1, 1, _BH), lambda m, e, h: (e, 0, h)),
            pl.BlockSpec((1, _BH, D), lambda m, e, h: (e, h, 0)),
            pl.BlockSpec((1, 1, D), lambda m, e, h: (e, 0, 0)),
        ],
        out_specs=pl.BlockSpec((_BM, D), lambda m, e, h: (m, 0)),
        out_shape=jax.ShapeDtypeStruct((T, D), jnp.float32),
        scratch_shapes=[pltpu.VMEM((_BM, _NE), jnp.float32)],
    )(x2, xb, Wg, bg2, W1b, b1_3, W2b, b2_3)
    return out.reshape(B, S, D)


# speedup vs baseline: 1.7756x; 1.7756x over previous
import jax
import jax.numpy as jnp
from jax import lax
from jax.experimental import pallas as pl
from jax.experimental.pallas import tpu as pltpu
from jax.experimental.pallas import tpu_sc as plsc

_NE = 8
_K = 2
_D = 1024
_H = 4096
_T = 4096
_BM = 512
_BH = 512
_P = _K * _T + _NE * _BM
_NB = _P // _BM
_NBPAD = 128
_NC = 2
_NS = 16
_NW = _NC * _NS
_CPW = _T // _NW
_CH = 32


def _gelu_exact(v):
    return 0.5 * v * (1.0 + lax.erf(v * 0.7071067811865476))


def _route_kernel(x_ref, Wg_ref, bg_ref, p1_ref, p2_ref, be_ref):
    scores = jnp.dot(x_ref[...], Wg_ref[...],
                     preferred_element_type=jnp.float32) + bg_ref[...]
    lane = lax.broadcasted_iota(jnp.int32, (_T, _NE), 1)
    m1 = jnp.max(scores, axis=-1, keepdims=True)
    f1 = jnp.min(jnp.where(scores == m1, lane, _NE), axis=-1, keepdims=True)
    rest = jnp.where(lane == f1, -jnp.inf, scores)
    m2 = jnp.max(rest, axis=-1, keepdims=True)
    f2 = jnp.min(jnp.where(rest == m2, lane, _NE), axis=-1, keepdims=True)
    oh1 = (lane == f1).astype(jnp.int32)
    oh2 = (lane == f2).astype(jnp.int32)

    def _cumsum_rows(a):
        k = 1
        while k < _T:
            a = a + jnp.concatenate(
                [jnp.zeros((k, _NE), jnp.int32), a[:_T - k, :]], axis=0)
            k *= 2
        return a

    incl1 = _cumsum_rows(oh1)
    incl2 = _cumsum_rows(oh2)
    excl1 = incl1 - oh1
    excl2 = incl2 - oh2
    cnt1 = incl1[_T - 1:_T, :]
    cnt2 = incl2[_T - 1:_T, :]
    counts = cnt1 + cnt2
    padded = jnp.bitwise_and(counts + (_BM - 1), ~(_BM - 1))
    inclp = padded
    for k in (1, 2, 4):
        inclp = inclp + jnp.concatenate(
            [jnp.zeros((1, k), jnp.int32), inclp[:, :_NE - k]], axis=1)
    pe = inclp - padded
    p1_ref[...] = jnp.sum(oh1 * (pe + excl1), axis=-1, keepdims=True)
    p2_ref[...] = jnp.sum(oh2 * (pe + cnt1 + excl2), axis=-1, keepdims=True)

    lane8 = lax.broadcasted_iota(jnp.int32, (1, _NE), 1)
    g_iota = lax.broadcasted_iota(jnp.int32, (1, _NBPAD), 1)
    be = jnp.zeros((1, _NBPAD), jnp.int32)
    for e in range(1, _NE):
        pe_e = jnp.sum(jnp.where(lane8 == e, pe, 0))
        be = be + (g_iota * _BM >= pe_e).astype(jnp.int32)
    total = jnp.sum(jnp.where(lane8 == _NE - 1, inclp, 0))
    be_ref[...] = jnp.where(g_iota * _BM < total, be, _NE)


def _scatter_body(x_hbm, p1_hbm, p2_hbm, xs_hbm, xbuf, i1, i2, sem):
    wid = lax.axis_index("s") * _NC + lax.axis_index("c")
    for c in range(_CPW // _CH):
        base = wid * _CPW + c * _CH
        pltpu.sync_copy(x_hbm.at[pl.ds(base, _CH)], xbuf)
        pltpu.sync_copy(p1_hbm.at[pl.ds(base, _CH)], i1)
        pltpu.sync_copy(p2_hbm.at[pl.ds(base, _CH)], i2)
        pltpu.async_copy(xbuf, xs_hbm.at[i1], sem).wait()
        pltpu.async_copy(xbuf, xs_hbm.at[i2], sem).wait()


def _gffn_kernel(be_ref, xs_ref, W1_ref, b1_ref, W2_ref, b2_ref, ys_ref):
    g = pl.program_id(0)
    h = pl.program_id(1)

    @pl.when(be_ref[g] != _NE)
    def _compute():
        hblk = _gelu_exact(
            jnp.dot(xs_ref[...], W1_ref[0],
                    preferred_element_type=jnp.float32) + b1_ref[0])
        contrib = jnp.dot(hblk, W2_ref[0], preferred_element_type=jnp.float32)

        @pl.when(h == 0)
        def _first():
            ys_ref[...] = contrib + b2_ref[0]

        @pl.when(h != 0)
        def _rest():
            ys_ref[...] += contrib


def _combine_body(ys_hbm, p1_hbm, p2_hbm, out_hbm, v1, v2, i1, i2, sem):
    wid = lax.axis_index("s") * _NC + lax.axis_index("c")
    for c in range(_CPW // _CH):
        base = wid * _CPW + c * _CH
        pltpu.sync_copy(p1_hbm.at[pl.ds(base, _CH)], i1)
        pltpu.sync_copy(p2_hbm.at[pl.ds(base, _CH)], i2)
        pltpu.async_copy(ys_hbm.at[i1], v1, sem).wait()
        pltpu.async_copy(ys_hbm.at[i2], v2, sem).wait()

        def _add(i, carry):
            r = i // (_D // 16)
            o = (i % (_D // 16)) * 16
            v1[r, pl.ds(o, 16)] = v1[r, pl.ds(o, 16)] + v2[r, pl.ds(o, 16)]
            return carry

        lax.fori_loop(0, _CH * (_D // 16), _add, 0)
        pltpu.sync_copy(v1, out_hbm.at[pl.ds(base, _CH)])


def _sc_mesh():
    return plsc.VectorSubcoreMesh(core_axis_name="c", subcore_axis_name="s")


def kernel(x, W1, b1, W2, b2, Wg, bg):
    B, S, D = x.shape
    x2 = x.reshape(_T, D)
    bg2 = bg.reshape(1, _NE)
    b1_3 = b1.reshape(_NE, 1, _H)
    b2_3 = b2.reshape(_NE, 1, _D)

    p1, p2, be = pl.pallas_call(
        _route_kernel,
        in_specs=[
            pl.BlockSpec((_T, _D), lambda: (0, 0)),
            pl.BlockSpec((_D, _NE), lambda: (0, 0)),
            pl.BlockSpec((1, _NE), lambda: (0, 0)),
        ],
        out_specs=[
            pl.BlockSpec((_T, 1), lambda: (0, 0)),
            pl.BlockSpec((_T, 1), lambda: (0, 0)),
            pl.BlockSpec((1, _NBPAD), lambda: (0, 0)),
        ],
        out_shape=[
            jax.ShapeDtypeStruct((_T, 1), jnp.int32),
            jax.ShapeDtypeStruct((_T, 1), jnp.int32),
            jax.ShapeDtypeStruct((1, _NBPAD), jnp.int32),
        ],
    )(x2, Wg, bg2)
    p1f = p1.reshape(_T)
    p2f = p2.reshape(_T)
    be1 = be.reshape(_NBPAD)

    xs = pl.kernel(
        _scatter_body,
        out_type=jax.ShapeDtypeStruct((_P, _D), jnp.float32),
        mesh=_sc_mesh(),
        scratch_types=[
            pltpu.VMEM((_CH, _D), jnp.float32),
            pltpu.VMEM((_CH,), jnp.int32),
            pltpu.VMEM((_CH,), jnp.int32),
            pltpu.SemaphoreType.DMA,
        ],
    )(x2, p1f, p2f)

    ys = pl.pallas_call(
        _gffn_kernel,
        grid_spec=pltpu.PrefetchScalarGridSpec(
            num_scalar_prefetch=1,
            grid=(_NB, _H // _BH),
            in_specs=[
                pl.BlockSpec((_BM, _D), lambda g, h, be: (g, 0)),
                pl.BlockSpec((1, _D, _BH),
                             lambda g, h, be: (jnp.minimum(be[g], _NE - 1), 0, h)),
                pl.BlockSpec((1, 1, _BH),
                             lambda g, h, be: (jnp.minimum(be[g], _NE - 1), 0, h)),
                pl.BlockSpec((1, _BH, _D),
                             lambda g, h, be: (jnp.minimum(be[g], _NE - 1), h, 0)),
                pl.BlockSpec((1, 1, _D),
                             lambda g, h, be: (jnp.minimum(be[g], _NE - 1), 0, 0)),
            ],
            out_specs=pl.BlockSpec((_BM, _D), lambda g, h, be: (g, 0)),
        ),
        out_shape=jax.ShapeDtypeStruct((_P, _D), jnp.float32),
    )(be1, xs, W1, b1_3, W2, b2_3)

    out2 = pl.kernel(
        _combine_body,
        out_type=jax.ShapeDtypeStruct((_T, _D), jnp.float32),
        mesh=_sc_mesh(),
        scratch_types=[
            pltpu.VMEM((_CH, _D), jnp.float32),
            pltpu.VMEM((_CH, _D), jnp.float32),
            pltpu.VMEM((_CH,), jnp.int32),
            pltpu.VMEM((_CH,), jnp.int32),
            pltpu.SemaphoreType.DMA,
        ],
    )(ys, p1f, p2f)
    return out2.reshape(B, S, D)


# speedup vs baseline: 1.8894x; 1.0641x over previous
import jax
import jax.numpy as jnp
from jax import lax
from jax.experimental import pallas as pl
from jax.experimental.pallas import tpu as pltpu
from jax.experimental.pallas import tpu_sc as plsc

_NE = 8
_K = 2
_D = 1024
_H = 4096
_T = 4096
_BM = 1024
_BH = 512
_P = _K * _T + _NE * _BM
_NB = _P // _BM
_NBPAD = 128
_NC = 2
_NS = 16
_NW = _NC * _NS
_CPW = _T // _NW
_CH = 32


def _gelu_exact(v):
    return 0.5 * v * (1.0 + lax.erf(v * 0.7071067811865476))


def _route_kernel(x_ref, Wg_ref, bg_ref, p1_ref, p2_ref, be_ref):
    scores = jnp.dot(x_ref[...], Wg_ref[...],
                     preferred_element_type=jnp.float32) + bg_ref[...]
    lane = lax.broadcasted_iota(jnp.int32, (_T, _NE), 1)
    m1 = jnp.max(scores, axis=-1, keepdims=True)
    f1 = jnp.min(jnp.where(scores == m1, lane, _NE), axis=-1, keepdims=True)
    rest = jnp.where(lane == f1, -jnp.inf, scores)
    m2 = jnp.max(rest, axis=-1, keepdims=True)
    f2 = jnp.min(jnp.where(rest == m2, lane, _NE), axis=-1, keepdims=True)
    oh1 = (lane == f1).astype(jnp.int32)
    oh2 = (lane == f2).astype(jnp.int32)

    def _cumsum_rows(a):
        k = 1
        while k < _T:
            a = a + jnp.concatenate(
                [jnp.zeros((k, _NE), jnp.int32), a[:_T - k, :]], axis=0)
            k *= 2
        return a

    incl1 = _cumsum_rows(oh1)
    incl2 = _cumsum_rows(oh2)
    excl1 = incl1 - oh1
    excl2 = incl2 - oh2
    cnt1 = incl1[_T - 1:_T, :]
    cnt2 = incl2[_T - 1:_T, :]
    counts = cnt1 + cnt2
    padded = jnp.bitwise_and(counts + (_BM - 1), ~(_BM - 1))
    inclp = padded
    for k in (1, 2, 4):
        inclp = inclp + jnp.concatenate(
            [jnp.zeros((1, k), jnp.int32), inclp[:, :_NE - k]], axis=1)
    pe = inclp - padded
    p1_ref[...] = jnp.sum(oh1 * (pe + excl1), axis=-1, keepdims=True)
    p2_ref[...] = jnp.sum(oh2 * (pe + cnt1 + excl2), axis=-1, keepdims=True)

    lane8 = lax.broadcasted_iota(jnp.int32, (1, _NE), 1)
    g_iota = lax.broadcasted_iota(jnp.int32, (1, _NBPAD), 1)
    be = jnp.zeros((1, _NBPAD), jnp.int32)
    for e in range(1, _NE):
        pe_e = jnp.sum(jnp.where(lane8 == e, pe, 0))
        be = be + (g_iota * _BM >= pe_e).astype(jnp.int32)
    total = jnp.sum(jnp.where(lane8 == _NE - 1, inclp, 0))
    be_ref[...] = jnp.where(g_iota * _BM < total, be, _NE)


def _scatter_body(x_hbm, p1_hbm, p2_hbm, xs_hbm, xbuf, i1, i2, sem, sem2):
    wid = lax.axis_index("s") * _NC + lax.axis_index("c")
    for c in range(_CPW // _CH):
        base = wid * _CPW + c * _CH
        dx = pltpu.async_copy(x_hbm.at[pl.ds(base, _CH)], xbuf, sem)
        d1 = pltpu.async_copy(p1_hbm.at[pl.ds(base, _CH)], i1, sem2)
        d2 = pltpu.async_copy(p2_hbm.at[pl.ds(base, _CH)], i2, sem2)
        dx.wait()
        d1.wait()
        d2.wait()
        s1 = pltpu.async_copy(xbuf, xs_hbm.at[i1], sem)
        s2 = pltpu.async_copy(xbuf, xs_hbm.at[i2], sem2)
        s1.wait()
        s2.wait()


def _gffn_kernel(be_ref, xs_ref, W1_ref, b1_ref, W2_ref, b2_ref, ys_ref):
    g = pl.program_id(0)
    h = pl.program_id(1)

    @pl.when(be_ref[g] != _NE)
    def _compute():
        hblk = _gelu_exact(
            jnp.dot(xs_ref[...], W1_ref[0],
                    preferred_element_type=jnp.float32) + b1_ref[0])
        contrib = jnp.dot(hblk, W2_ref[0], preferred_element_type=jnp.float32)

        @pl.when(h == 0)
        def _first():
            ys_ref[...] = contrib + b2_ref[0]

        @pl.when(h != 0)
        def _rest():
            ys_ref[...] += contrib


def _combine_body(ys_hbm, p1_hbm, p2_hbm, out_hbm, v1, v2, i1, i2, sem, sem2):
    wid = lax.axis_index("s") * _NC + lax.axis_index("c")
    for c in range(_CPW // _CH):
        base = wid * _CPW + c * _CH
        d1 = pltpu.async_copy(p1_hbm.at[pl.ds(base, _CH)], i1, sem)
        d2 = pltpu.async_copy(p2_hbm.at[pl.ds(base, _CH)], i2, sem2)
        d1.wait()
        d2.wait()
        g1 = pltpu.async_copy(ys_hbm.at[i1], v1, sem)
        g2 = pltpu.async_copy(ys_hbm.at[i2], v2, sem2)
        g1.wait()
        g2.wait()

        @plsc.parallel_loop(0, _CH * (_D // 16), 1)
        def _add(i):
            r = i // (_D // 16)
            o = (i % (_D // 16)) * 16
            v1[r, pl.ds(o, 16)] = v1[r, pl.ds(o, 16)] + v2[r, pl.ds(o, 16)]

        pltpu.sync_copy(v1, out_hbm.at[pl.ds(base, _CH)])


def _sc_mesh():
    return plsc.VectorSubcoreMesh(core_axis_name="c", subcore_axis_name="s")


def kernel(x, W1, b1, W2, b2, Wg, bg):
    B, S, D = x.shape
    x2 = x.reshape(_T, D)
    bg2 = bg.reshape(1, _NE)
    b1_3 = b1.reshape(_NE, 1, _H)
    b2_3 = b2.reshape(_NE, 1, _D)

    p1, p2, be = pl.pallas_call(
        _route_kernel,
        in_specs=[
            pl.BlockSpec((_T, _D), lambda: (0, 0)),
            pl.BlockSpec((_D, _NE), lambda: (0, 0)),
            pl.BlockSpec((1, _NE), lambda: (0, 0)),
        ],
        out_specs=[
            pl.BlockSpec((_T, 1), lambda: (0, 0)),
            pl.BlockSpec((_T, 1), lambda: (0, 0)),
            pl.BlockSpec((1, _NBPAD), lambda: (0, 0)),
        ],
        out_shape=[
            jax.ShapeDtypeStruct((_T, 1), jnp.int32),
            jax.ShapeDtypeStruct((_T, 1), jnp.int32),
            jax.ShapeDtypeStruct((1, _NBPAD), jnp.int32),
        ],
    )(x2, Wg, bg2)
    p1f = p1.reshape(_T)
    p2f = p2.reshape(_T)
    be1 = be.reshape(_NBPAD)

    xs = pl.kernel(
        _scatter_body,
        out_type=jax.ShapeDtypeStruct((_P, _D), jnp.float32),
        mesh=_sc_mesh(),
        scratch_types=[
            pltpu.VMEM((_CH, _D), jnp.float32),
            pltpu.VMEM((_CH,), jnp.int32),
            pltpu.VMEM((_CH,), jnp.int32),
            pltpu.SemaphoreType.DMA,
            pltpu.SemaphoreType.DMA,
        ],
    )(x2, p1f, p2f)

    ys = pl.pallas_call(
        _gffn_kernel,
        grid_spec=pltpu.PrefetchScalarGridSpec(
            num_scalar_prefetch=1,
            grid=(_NB, _H // _BH),
            in_specs=[
                pl.BlockSpec((_BM, _D), lambda g, h, be: (g, 0)),
                pl.BlockSpec((1, _D, _BH),
                             lambda g, h, be: (jnp.minimum(be[g], _NE - 1), 0, h)),
                pl.BlockSpec((1, 1, _BH),
                             lambda g, h, be: (jnp.minimum(be[g], _NE - 1), 0, h)),
                pl.BlockSpec((1, _BH, _D),
                             lambda g, h, be: (jnp.minimum(be[g], _NE - 1), h, 0)),
                pl.BlockSpec((1, 1, _D),
                             lambda g, h, be: (jnp.minimum(be[g], _NE - 1), 0, 0)),
            ],
            out_specs=pl.BlockSpec((_BM, _D), lambda g, h, be: (g, 0)),
        ),
        out_shape=jax.ShapeDtypeStruct((_P, _D), jnp.float32),
    )(be1, xs, W1, b1_3, W2, b2_3)

    out2 = pl.kernel(
        _combine_body,
        out_type=jax.ShapeDtypeStruct((_T, _D), jnp.float32),
        mesh=_sc_mesh(),
        scratch_types=[
            pltpu.VMEM((_CH, _D), jnp.float32),
            pltpu.VMEM((_CH, _D), jnp.float32),
            pltpu.VMEM((_CH,), jnp.int32),
            pltpu.VMEM((_CH,), jnp.int32),
            pltpu.SemaphoreType.DMA,
            pltpu.SemaphoreType.DMA,
        ],
    )(ys, p1f, p2f)
    return out2.reshape(B, S, D)


# speedup vs baseline: 2.1299x; 1.1273x over previous
import jax
import jax.numpy as jnp
from jax import lax
from jax.experimental import pallas as pl
from jax.experimental.pallas import tpu as pltpu
from jax.experimental.pallas import tpu_sc as plsc

_NE = 8
_K = 2
_D = 1024
_H = 4096
_T = 4096
_BM = 1024
_BH = 1024
_P = _K * _T + _NE * _BM
_NB = _P // _BM
_NBPAD = 128
_NC = 2
_NS = 16
_NW = _NC * _NS
_CPW = _T // _NW
_CH = 32


def _gelu_exact(v):
    return 0.5 * v * (1.0 + lax.erf(v * 0.7071067811865476))


def _route_kernel(x_ref, Wg_ref, bg_ref, p1_ref, p2_ref, be_ref):
    scores = jnp.dot(x_ref[...], Wg_ref[...],
                     preferred_element_type=jnp.float32) + bg_ref[...]
    lane = lax.broadcasted_iota(jnp.int32, (_T, _NE), 1)
    m1 = jnp.max(scores, axis=-1, keepdims=True)
    f1 = jnp.min(jnp.where(scores == m1, lane, _NE), axis=-1, keepdims=True)
    rest = jnp.where(lane == f1, -jnp.inf, scores)
    m2 = jnp.max(rest, axis=-1, keepdims=True)
    f2 = jnp.min(jnp.where(rest == m2, lane, _NE), axis=-1, keepdims=True)
    oh1 = (lane == f1).astype(jnp.int32)
    oh2 = (lane == f2).astype(jnp.int32)

    def _cumsum_rows(a):
        k = 1
        while k < _T:
            a = a + jnp.concatenate(
                [jnp.zeros((k, _NE), jnp.int32), a[:_T - k, :]], axis=0)
            k *= 2
        return a

    incl1 = _cumsum_rows(oh1)
    incl2 = _cumsum_rows(oh2)
    excl1 = incl1 - oh1
    excl2 = incl2 - oh2
    cnt1 = incl1[_T - 1:_T, :]
    cnt2 = incl2[_T - 1:_T, :]
    counts = cnt1 + cnt2
    padded = jnp.bitwise_and(counts + (_BM - 1), ~(_BM - 1))
    inclp = padded
    for k in (1, 2, 4):
        inclp = inclp + jnp.concatenate(
            [jnp.zeros((1, k), jnp.int32), inclp[:, :_NE - k]], axis=1)
    pe = inclp - padded
    p1_ref[...] = jnp.sum(oh1 * (pe + excl1), axis=-1, keepdims=True)
    p2_ref[...] = jnp.sum(oh2 * (pe + cnt1 + excl2), axis=-1, keepdims=True)

    lane8 = lax.broadcasted_iota(jnp.int32, (1, _NE), 1)
    g_iota = lax.broadcasted_iota(jnp.int32, (1, _NBPAD), 1)
    be = jnp.zeros((1, _NBPAD), jnp.int32)
    for e in range(1, _NE):
        pe_e = jnp.sum(jnp.where(lane8 == e, pe, 0))
        be = be + (g_iota * _BM >= pe_e).astype(jnp.int32)
    total = jnp.sum(jnp.where(lane8 == _NE - 1, inclp, 0))
    be_ref[...] = jnp.where(g_iota * _BM < total, be, _NE)


def _scatter_body(x_hbm, p1_hbm, p2_hbm, xs_hbm, xbuf, i1, i2, sem, sem2):
    wid = lax.axis_index("s") * _NC + lax.axis_index("c")
    for c in range(_CPW // _CH):
        base = wid * _CPW + c * _CH
        dx = pltpu.async_copy(x_hbm.at[pl.ds(base, _CH)], xbuf, sem)
        d1 = pltpu.async_copy(p1_hbm.at[pl.ds(base, _CH)], i1, sem2)
        d2 = pltpu.async_copy(p2_hbm.at[pl.ds(base, _CH)], i2, sem2)
        dx.wait()
        d1.wait()
        d2.wait()
        s1 = pltpu.async_copy(xbuf, xs_hbm.at[i1], sem)
        s2 = pltpu.async_copy(xbuf, xs_hbm.at[i2], sem2)
        s1.wait()
        s2.wait()


def _gffn_kernel(be_ref, xs_ref, W1_ref, b1_ref, W2_ref, b2_ref, ys_ref):
    g = pl.program_id(0)
    h = pl.program_id(1)

    @pl.when(be_ref[g] != _NE)
    def _compute():
        hblk = _gelu_exact(
            jnp.dot(xs_ref[...], W1_ref[0],
                    preferred_element_type=jnp.float32) + b1_ref[0])
        contrib = jnp.dot(hblk, W2_ref[0], preferred_element_type=jnp.float32)

        @pl.when(h == 0)
        def _first():
            ys_ref[...] = contrib + b2_ref[0]

        @pl.when(h != 0)
        def _rest():
            ys_ref[...] += contrib


def _combine_body(ys_hbm, p1_hbm, p2_hbm, out_hbm, v1, v2, i1, i2, sem, sem2):
    wid = lax.axis_index("s") * _NC + lax.axis_index("c")
    for c in range(_CPW // _CH):
        base = wid * _CPW + c * _CH
        d1 = pltpu.async_copy(p1_hbm.at[pl.ds(base, _CH)], i1, sem)
        d2 = pltpu.async_copy(p2_hbm.at[pl.ds(base, _CH)], i2, sem2)
        d1.wait()
        d2.wait()
        g1 = pltpu.async_copy(ys_hbm.at[i1], v1, sem)
        g2 = pltpu.async_copy(ys_hbm.at[i2], v2, sem2)
        g1.wait()
        g2.wait()

        @plsc.parallel_loop(0, _CH * (_D // 16), 1)
        def _add(i):
            r = i // (_D // 16)
            o = (i % (_D // 16)) * 16
            v1[r, pl.ds(o, 16)] = v1[r, pl.ds(o, 16)] + v2[r, pl.ds(o, 16)]

        pltpu.sync_copy(v1, out_hbm.at[pl.ds(base, _CH)])


def _sc_mesh():
    return plsc.VectorSubcoreMesh(core_axis_name="c", subcore_axis_name="s")


def kernel(x, W1, b1, W2, b2, Wg, bg):
    B, S, D = x.shape
    x2 = x.reshape(_T, D)
    bg2 = bg.reshape(1, _NE)
    b1_3 = b1.reshape(_NE, 1, _H)
    b2_3 = b2.reshape(_NE, 1, _D)

    p1, p2, be = pl.pallas_call(
        _route_kernel,
        in_specs=[
            pl.BlockSpec((_T, _D), lambda: (0, 0)),
            pl.BlockSpec((_D, _NE), lambda: (0, 0)),
            pl.BlockSpec((1, _NE), lambda: (0, 0)),
        ],
        out_specs=[
            pl.BlockSpec((_T, 1), lambda: (0, 0)),
            pl.BlockSpec((_T, 1), lambda: (0, 0)),
            pl.BlockSpec((1, _NBPAD), lambda: (0, 0)),
        ],
        out_shape=[
            jax.ShapeDtypeStruct((_T, 1), jnp.int32),
            jax.ShapeDtypeStruct((_T, 1), jnp.int32),
            jax.ShapeDtypeStruct((1, _NBPAD), jnp.int32),
        ],
    )(x2, Wg, bg2)
    p1f = p1.reshape(_T)
    p2f = p2.reshape(_T)
    be1 = be.reshape(_NBPAD)

    xs = pl.kernel(
        _scatter_body,
        out_type=jax.ShapeDtypeStruct((_P, _D), jnp.float32),
        mesh=_sc_mesh(),
        scratch_types=[
            pltpu.VMEM((_CH, _D), jnp.float32),
            pltpu.VMEM((_CH,), jnp.int32),
            pltpu.VMEM((_CH,), jnp.int32),
            pltpu.SemaphoreType.DMA,
            pltpu.SemaphoreType.DMA,
        ],
    )(x2, p1f, p2f)

    ys = pl.pallas_call(
        _gffn_kernel,
        grid_spec=pltpu.PrefetchScalarGridSpec(
            num_scalar_prefetch=1,
            grid=(_NB, _H // _BH),
            in_specs=[
                pl.BlockSpec((_BM, _D), lambda g, h, be: (g, 0)),
                pl.BlockSpec((1, _D, _BH),
                             lambda g, h, be: (jnp.minimum(be[g], _NE - 1), 0, h)),
                pl.BlockSpec((1, 1, _BH),
                             lambda g, h, be: (jnp.minimum(be[g], _NE - 1), 0, h)),
                pl.BlockSpec((1, _BH, _D),
                             lambda g, h, be: (jnp.minimum(be[g], _NE - 1), h, 0)),
                pl.BlockSpec((1, 1, _D),
                             lambda g, h, be: (jnp.minimum(be[g], _NE - 1), 0, 0)),
            ],
            out_specs=pl.BlockSpec((_BM, _D), lambda g, h, be: (g, 0)),
        ),
        out_shape=jax.ShapeDtypeStruct((_P, _D), jnp.float32),
    )(be1, xs, W1, b1_3, W2, b2_3)

    out2 = pl.kernel(
        _combine_body,
        out_type=jax.ShapeDtypeStruct((_T, _D), jnp.float32),
        mesh=_sc_mesh(),
        scratch_types=[
            pltpu.VMEM((_CH, _D), jnp.float32),
            pltpu.VMEM((_CH, _D), jnp.float32),
            pltpu.VMEM((_CH,), jnp.int32),
            pltpu.VMEM((_CH,), jnp.int32),
            pltpu.SemaphoreType.DMA,
            pltpu.SemaphoreType.DMA,
        ],
    )(ys, p1f, p2f)
    return out2.reshape(B, S, D)


# speedup vs baseline: 2.2315x; 1.0477x over previous
import jax
import jax.numpy as jnp
from jax import lax
from jax.experimental import pallas as pl
from jax.experimental.pallas import tpu as pltpu
from jax.experimental.pallas import tpu_sc as plsc

_NE = 8
_K = 2
_D = 1024
_H = 4096
_T = 4096
_BM = 1024
_BH = 2048
_P = _K * _T + _NE * _BM
_NB = _P // _BM
_NBPAD = 128
_NC = 2
_NS = 16
_NW = _NC * _NS
_CPW = _T // _NW
_CH = 32


def _gelu_exact(v):
    return 0.5 * v * (1.0 + lax.erf(v * 0.7071067811865476))


def _route_kernel(x_ref, Wg_ref, bg_ref, p1_ref, p2_ref, be_ref):
    scores = jnp.dot(x_ref[...], Wg_ref[...],
                     preferred_element_type=jnp.float32) + bg_ref[...]
    lane = lax.broadcasted_iota(jnp.int32, (_T, _NE), 1)
    m1 = jnp.max(scores, axis=-1, keepdims=True)
    f1 = jnp.min(jnp.where(scores == m1, lane, _NE), axis=-1, keepdims=True)
    rest = jnp.where(lane == f1, -jnp.inf, scores)
    m2 = jnp.max(rest, axis=-1, keepdims=True)
    f2 = jnp.min(jnp.where(rest == m2, lane, _NE), axis=-1, keepdims=True)
    oh1 = (lane == f1).astype(jnp.int32)
    oh2 = (lane == f2).astype(jnp.int32)

    def _cumsum_rows(a):
        k = 1
        while k < _T:
            a = a + jnp.concatenate(
                [jnp.zeros((k, _NE), jnp.int32), a[:_T - k, :]], axis=0)
            k *= 2
        return a

    incl1 = _cumsum_rows(oh1)
    incl2 = _cumsum_rows(oh2)
    excl1 = incl1 - oh1
    excl2 = incl2 - oh2
    cnt1 = incl1[_T - 1:_T, :]
    cnt2 = incl2[_T - 1:_T, :]
    counts = cnt1 + cnt2
    padded = jnp.bitwise_and(counts + (_BM - 1), ~(_BM - 1))
    inclp = padded
    for k in (1, 2, 4):
        inclp = inclp + jnp.concatenate(
            [jnp.zeros((1, k), jnp.int32), inclp[:, :_NE - k]], axis=1)
    pe = inclp - padded
    p1_ref[...] = jnp.sum(oh1 * (pe + excl1), axis=-1, keepdims=True)
    p2_ref[...] = jnp.sum(oh2 * (pe + cnt1 + excl2), axis=-1, keepdims=True)

    lane8 = lax.broadcasted_iota(jnp.int32, (1, _NE), 1)
    g_iota = lax.broadcasted_iota(jnp.int32, (1, _NBPAD), 1)
    be = jnp.zeros((1, _NBPAD), jnp.int32)
    for e in range(1, _NE):
        pe_e = jnp.sum(jnp.where(lane8 == e, pe, 0))
        be = be + (g_iota * _BM >= pe_e).astype(jnp.int32)
    total = jnp.sum(jnp.where(lane8 == _NE - 1, inclp, 0))
    be_ref[...] = jnp.where(g_iota * _BM < total, be, _NE)


def _scatter_body(x_hbm, p1_hbm, p2_hbm, xs_hbm, xbuf, i1, i2, sem, sem2):
    wid = lax.axis_index("s") * _NC + lax.axis_index("c")
    for c in range(_CPW // _CH):
        base = wid * _CPW + c * _CH
        dx = pltpu.async_copy(x_hbm.at[pl.ds(base, _CH)], xbuf, sem)
        d1 = pltpu.async_copy(p1_hbm.at[pl.ds(base, _CH)], i1, sem2)
        d2 = pltpu.async_copy(p2_hbm.at[pl.ds(base, _CH)], i2, sem2)
        dx.wait()
        d1.wait()
        d2.wait()
        s1 = pltpu.async_copy(xbuf, xs_hbm.at[i1], sem)
        s2 = pltpu.async_copy(xbuf, xs_hbm.at[i2], sem2)
        s1.wait()
        s2.wait()


def _gffn_kernel(be_ref, xs_ref, W1_ref, b1_ref, W2_ref, b2_ref, ys_ref):
    g = pl.program_id(0)
    h = pl.program_id(1)

    @pl.when(be_ref[g] != _NE)
    def _compute():
        hblk = _gelu_exact(
            jnp.dot(xs_ref[...], W1_ref[0],
                    preferred_element_type=jnp.float32) + b1_ref[0])
        contrib = jnp.dot(hblk, W2_ref[0], preferred_element_type=jnp.float32)

        @pl.when(h == 0)
        def _first():
            ys_ref[...] = contrib + b2_ref[0]

        @pl.when(h != 0)
        def _rest():
            ys_ref[...] += contrib


def _combine_body(ys_hbm, p1_hbm, p2_hbm, out_hbm, v1, v2, i1, i2, sem, sem2):
    wid = lax.axis_index("s") * _NC + lax.axis_index("c")
    for c in range(_CPW // _CH):
        base = wid * _CPW + c * _CH
        d1 = pltpu.async_copy(p1_hbm.at[pl.ds(base, _CH)], i1, sem)
        d2 = pltpu.async_copy(p2_hbm.at[pl.ds(base, _CH)], i2, sem2)
        d1.wait()
        d2.wait()
        g1 = pltpu.async_copy(ys_hbm.at[i1], v1, sem)
        g2 = pltpu.async_copy(ys_hbm.at[i2], v2, sem2)
        g1.wait()
        g2.wait()

        @plsc.parallel_loop(0, _CH * (_D // 16), 1)
        def _add(i):
            r = i // (_D // 16)
            o = (i % (_D // 16)) * 16
            v1[r, pl.ds(o, 16)] = v1[r, pl.ds(o, 16)] + v2[r, pl.ds(o, 16)]

        pltpu.sync_copy(v1, out_hbm.at[pl.ds(base, _CH)])


def _sc_mesh():
    return plsc.VectorSubcoreMesh(core_axis_name="c", subcore_axis_name="s")


def kernel(x, W1, b1, W2, b2, Wg, bg):
    B, S, D = x.shape
    x2 = x.reshape(_T, D)
    bg2 = bg.reshape(1, _NE)
    b1_3 = b1.reshape(_NE, 1, _H)
    b2_3 = b2.reshape(_NE, 1, _D)

    p1, p2, be = pl.pallas_call(
        _route_kernel,
        in_specs=[
            pl.BlockSpec((_T, _D), lambda: (0, 0)),
            pl.BlockSpec((_D, _NE), lambda: (0, 0)),
            pl.BlockSpec((1, _NE), lambda: (0, 0)),
        ],
        out_specs=[
            pl.BlockSpec((_T, 1), lambda: (0, 0)),
            pl.BlockSpec((_T, 1), lambda: (0, 0)),
            pl.BlockSpec((1, _NBPAD), lambda: (0, 0)),
        ],
        out_shape=[
            jax.ShapeDtypeStruct((_T, 1), jnp.int32),
            jax.ShapeDtypeStruct((_T, 1), jnp.int32),
            jax.ShapeDtypeStruct((1, _NBPAD), jnp.int32),
        ],
    )(x2, Wg, bg2)
    p1f = p1.reshape(_T)
    p2f = p2.reshape(_T)
    be1 = be.reshape(_NBPAD)

    xs = pl.kernel(
        _scatter_body,
        out_type=jax.ShapeDtypeStruct((_P, _D), jnp.float32),
        mesh=_sc_mesh(),
        scratch_types=[
            pltpu.VMEM((_CH, _D), jnp.float32),
            pltpu.VMEM((_CH,), jnp.int32),
            pltpu.VMEM((_CH,), jnp.int32),
            pltpu.SemaphoreType.DMA,
            pltpu.SemaphoreType.DMA,
        ],
    )(x2, p1f, p2f)

    ys = pl.pallas_call(
        _gffn_kernel,
        grid_spec=pltpu.PrefetchScalarGridSpec(
            num_scalar_prefetch=1,
            grid=(_NB, _H // _BH),
            in_specs=[
                pl.BlockSpec((_BM, _D), lambda g, h, be: (g, 0)),
                pl.BlockSpec((1, _D, _BH),
                             lambda g, h, be: (jnp.minimum(be[g], _NE - 1), 0, h)),
                pl.BlockSpec((1, 1, _BH),
                             lambda g, h, be: (jnp.minimum(be[g], _NE - 1), 0, h)),
                pl.BlockSpec((1, _BH, _D),
                             lambda g, h, be: (jnp.minimum(be[g], _NE - 1), h, 0)),
                pl.BlockSpec((1, 1, _D),
                             lambda g, h, be: (jnp.minimum(be[g], _NE - 1), 0, 0)),
            ],
            out_specs=pl.BlockSpec((_BM, _D), lambda g, h, be: (g, 0)),
        ),
        out_shape=jax.ShapeDtypeStruct((_P, _D), jnp.float32),
    )(be1, xs, W1, b1_3, W2, b2_3)

    out2 = pl.kernel(
        _combine_body,
        out_type=jax.ShapeDtypeStruct((_T, _D), jnp.float32),
        mesh=_sc_mesh(),
        scratch_types=[
            pltpu.VMEM((_CH, _D), jnp.float32),
            pltpu.VMEM((_CH, _D), jnp.float32),
            pltpu.VMEM((_CH,), jnp.int32),
            pltpu.VMEM((_CH,), jnp.int32),
            pltpu.SemaphoreType.DMA,
            pltpu.SemaphoreType.DMA,
        ],
    )(ys, p1f, p2f)
    return out2.reshape(B, S, D)


# speedup vs baseline: 2.2738x; 1.0190x over previous
import jax
import jax.numpy as jnp
from jax import lax
from jax.experimental import pallas as pl
from jax.experimental.pallas import tpu as pltpu
from jax.experimental.pallas import tpu_sc as plsc

_NE = 8
_K = 2
_D = 1024
_H = 4096
_T = 4096
_BM = 1024
_BH = 2048
_P = _K * _T + _NE * _BM
_NB = _P // _BM
_NBPAD = 128
_NC = 2
_NS = 16
_NW = _NC * _NS
_CPW = _T // _NW
_CH = 32
_CHC = 16


def _gelu_exact(v):
    return 0.5 * v * (1.0 + lax.erf(v * 0.7071067811865476))


def _route_kernel(x_ref, Wg_ref, bg_ref, p1_ref, p2_ref, be_ref):
    scores = jnp.dot(x_ref[...], Wg_ref[...],
                     preferred_element_type=jnp.float32) + bg_ref[...]
    lane = lax.broadcasted_iota(jnp.int32, (_T, _NE), 1)
    m1 = jnp.max(scores, axis=-1, keepdims=True)
    f1 = jnp.min(jnp.where(scores == m1, lane, _NE), axis=-1, keepdims=True)
    rest = jnp.where(lane == f1, -jnp.inf, scores)
    m2 = jnp.max(rest, axis=-1, keepdims=True)
    f2 = jnp.min(jnp.where(rest == m2, lane, _NE), axis=-1, keepdims=True)
    oh1 = (lane == f1).astype(jnp.int32)
    oh2 = (lane == f2).astype(jnp.int32)

    def _cumsum_rows(a):
        k = 1
        while k < _T:
            a = a + jnp.concatenate(
                [jnp.zeros((k, _NE), jnp.int32), a[:_T - k, :]], axis=0)
            k *= 2
        return a

    incl1 = _cumsum_rows(oh1)
    incl2 = _cumsum_rows(oh2)
    excl1 = incl1 - oh1
    excl2 = incl2 - oh2
    cnt1 = incl1[_T - 1:_T, :]
    cnt2 = incl2[_T - 1:_T, :]
    counts = cnt1 + cnt2
    padded = jnp.bitwise_and(counts + (_BM - 1), ~(_BM - 1))
    inclp = padded
    for k in (1, 2, 4):
        inclp = inclp + jnp.concatenate(
            [jnp.zeros((1, k), jnp.int32), inclp[:, :_NE - k]], axis=1)
    pe = inclp - padded
    p1_ref[...] = jnp.sum(oh1 * (pe + excl1), axis=-1, keepdims=True)
    p2_ref[...] = jnp.sum(oh2 * (pe + cnt1 + excl2), axis=-1, keepdims=True)

    lane8 = lax.broadcasted_iota(jnp.int32, (1, _NE), 1)
    g_iota = lax.broadcasted_iota(jnp.int32, (1, _NBPAD), 1)
    be = jnp.zeros((1, _NBPAD), jnp.int32)
    for e in range(1, _NE):
        pe_e = jnp.sum(jnp.where(lane8 == e, pe, 0))
        be = be + (g_iota * _BM >= pe_e).astype(jnp.int32)
    total = jnp.sum(jnp.where(lane8 == _NE - 1, inclp, 0))
    be_ref[...] = jnp.where(g_iota * _BM < total, be, _NE)


def _scatter_body(x_hbm, p1_hbm, p2_hbm, xs_hbm, xbuf, i1, i2, sem, sem2):
    wid = lax.axis_index("s") * _NC + lax.axis_index("c")
    for c in range(_CPW // _CH):
        base = wid * _CPW + c * _CH
        dx = pltpu.async_copy(x_hbm.at[pl.ds(base, _CH)], xbuf, sem)
        d1 = pltpu.async_copy(p1_hbm.at[pl.ds(base, _CH)], i1, sem2)
        d2 = pltpu.async_copy(p2_hbm.at[pl.ds(base, _CH)], i2, sem2)
        dx.wait()
        d1.wait()
        d2.wait()
        s1 = pltpu.async_copy(xbuf, xs_hbm.at[i1], sem)
        s2 = pltpu.async_copy(xbuf, xs_hbm.at[i2], sem2)
        s1.wait()
        s2.wait()


def _gffn_kernel(be_ref, xs_ref, W1_ref, b1_ref, W2_ref, b2_ref, ys_ref):
    g = pl.program_id(0)
    h = pl.program_id(1)

    @pl.when(be_ref[g] != _NE)
    def _compute():
        hblk = _gelu_exact(
            jnp.dot(xs_ref[...], W1_ref[0],
                    preferred_element_type=jnp.float32) + b1_ref[0])
        contrib = jnp.dot(hblk, W2_ref[0], preferred_element_type=jnp.float32)

        @pl.when(h == 0)
        def _first():
            ys_ref[...] = contrib + b2_ref[0]

        @pl.when(h != 0)
        def _rest():
            ys_ref[...] += contrib


def _combine_body(ys_hbm, p1_hbm, p2_hbm, out_hbm,
                  v1a, v1b, v2a, v2b, i1a, i1b, i2a, i2b,
                  sa1, sb1, sa2, sb2):
    wid = lax.axis_index("s") * _NC + lax.axis_index("c")
    v1 = (v1a, v1b)
    v2 = (v2a, v2b)
    i1 = (i1a, i1b)
    i2 = (i2a, i2b)
    s1 = (sa1, sb1)
    s2 = (sa2, sb2)
    nch = _CPW // _CHC

    def _fire(c):
        p = c % 2
        base = wid * _CPW + c * _CHC
        pltpu.sync_copy(p1_hbm.at[pl.ds(base, _CHC)], i1[p])
        pltpu.sync_copy(p2_hbm.at[pl.ds(base, _CHC)], i2[p])
        return (pltpu.async_copy(ys_hbm.at[i1[p]], v1[p], s1[p]),
                pltpu.async_copy(ys_hbm.at[i2[p]], v2[p], s2[p]))

    pending = _fire(0)
    for c in range(nch):
        p = c % 2
        da, db = pending
        if c + 1 < nch:
            nxt = _fire(c + 1)
        da.wait()
        db.wait()

        @plsc.parallel_loop(0, _CHC * (_D // 16), 1)
        def _add(i):
            r = i // (_D // 16)
            o = (i % (_D // 16)) * 16
            v1[p][r, pl.ds(o, 16)] = (v1[p][r, pl.ds(o, 16)] +
                                      v2[p][r, pl.ds(o, 16)])

        base = wid * _CPW + c * _CHC
        pltpu.sync_copy(v1[p], out_hbm.at[pl.ds(base, _CHC)])
        if c + 1 < nch:
            pending = nxt


def _sc_mesh():
    return plsc.VectorSubcoreMesh(core_axis_name="c", subcore_axis_name="s")


def kernel(x, W1, b1, W2, b2, Wg, bg):
    B, S, D = x.shape
    x2 = x.reshape(_T, D)
    bg2 = bg.reshape(1, _NE)
    b1_3 = b1.reshape(_NE, 1, _H)
    b2_3 = b2.reshape(_NE, 1, _D)

    p1, p2, be = pl.pallas_call(
        _route_kernel,
        in_specs=[
            pl.BlockSpec((_T, _D), lambda: (0, 0)),
            pl.BlockSpec((_D, _NE), lambda: (0, 0)),
            pl.BlockSpec((1, _NE), lambda: (0, 0)),
        ],
        out_specs=[
            pl.BlockSpec((_T, 1), lambda: (0, 0)),
            pl.BlockSpec((_T, 1), lambda: (0, 0)),
            pl.BlockSpec((1, _NBPAD), lambda: (0, 0)),
        ],
        out_shape=[
            jax.ShapeDtypeStruct((_T, 1), jnp.int32),
            jax.ShapeDtypeStruct((_T, 1), jnp.int32),
            jax.ShapeDtypeStruct((1, _NBPAD), jnp.int32),
        ],
    )(x2, Wg, bg2)
    p1f = p1.reshape(_T)
    p2f = p2.reshape(_T)
    be1 = be.reshape(_NBPAD)

    xs = pl.kernel(
        _scatter_body,
        out_type=jax.ShapeDtypeStruct((_P, _D), jnp.float32),
        mesh=_sc_mesh(),
        scratch_types=[
            pltpu.VMEM((_CH, _D), jnp.float32),
            pltpu.VMEM((_CH,), jnp.int32),
            pltpu.VMEM((_CH,), jnp.int32),
            pltpu.SemaphoreType.DMA,
            pltpu.SemaphoreType.DMA,
        ],
    )(x2, p1f, p2f)

    ys = pl.pallas_call(
        _gffn_kernel,
        grid_spec=pltpu.PrefetchScalarGridSpec(
            num_scalar_prefetch=1,
            grid=(_NB, _H // _BH),
            in_specs=[
                pl.BlockSpec((_BM, _D), lambda g, h, be: (g, 0)),
                pl.BlockSpec((1, _D, _BH),
                             lambda g, h, be: (jnp.minimum(be[g], _NE - 1), 0, h)),
                pl.BlockSpec((1, 1, _BH),
                             lambda g, h, be: (jnp.minimum(be[g], _NE - 1), 0, h)),
                pl.BlockSpec((1, _BH, _D),
                             lambda g, h, be: (jnp.minimum(be[g], _NE - 1), h, 0)),
                pl.BlockSpec((1, 1, _D),
                             lambda g, h, be: (jnp.minimum(be[g], _NE - 1), 0, 0)),
            ],
            out_specs=pl.BlockSpec((_BM, _D), lambda g, h, be: (g, 0)),
        ),
        out_shape=jax.ShapeDtypeStruct((_P, _D), jnp.float32),
    )(be1, xs, W1, b1_3, W2, b2_3)

    out2 = pl.kernel(
        _combine_body,
        out_type=jax.ShapeDtypeStruct((_T, _D), jnp.float32),
        mesh=_sc_mesh(),
        scratch_types=[
            pltpu.VMEM((_CHC, _D), jnp.float32),
            pltpu.VMEM((_CHC, _D), jnp.float32),
            pltpu.VMEM((_CHC, _D), jnp.float32),
            pltpu.VMEM((_CHC, _D), jnp.float32),
            pltpu.VMEM((_CHC,), jnp.int32),
            pltpu.VMEM((_CHC,), jnp.int32),
            pltpu.VMEM((_CHC,), jnp.int32),
            pltpu.VMEM((_CHC,), jnp.int32),
            pltpu.SemaphoreType.DMA,
            pltpu.SemaphoreType.DMA,
            pltpu.SemaphoreType.DMA,
            pltpu.SemaphoreType.DMA,
        ],
    )(ys, p1f, p2f)
    return out2.reshape(B, S, D)
